# baseline (speedup 1.0000x reference)
import functools

import jax
import jax.numpy as jnp
from jax import lax
from jax.experimental import pallas as pl
from jax.experimental.pallas import tpu as pltpu
from jax.experimental.pallas import tpu_sc as plsc

_B = 128
_S = 128
_N0, _N1, _N2, _N3 = 2048, 1024, 512, 256
_C0, _C1, _C2, _C3, _C4 = 64, 128, 128, 256, 256
_M = _B * _S
_EPS = 1e-5
_NC, _NS = 2, 16
_NW = _NC * _NS
_BPW = _B // _NW
_ABLK = 8
_BLK = 2048
_CH = 64

_SC_PARAMS = pltpu.CompilerParams(needs_layout_passes=False)


def _sc_mesh():
    return plsc.VectorSubcoreMesh(
        core_axis_name="c", subcore_axis_name="s",
        num_cores=_NC, num_subcores=_NS)


def _sc_chain():
    @functools.partial(
        pl.kernel,
        mesh=_sc_mesh(),
        compiler_params=_SC_PARAMS,
        out_type=(
            jax.ShapeDtypeStruct((_B, _S), jnp.int32),
            jax.ShapeDtypeStruct((_B, _S), jnp.int32),
            jax.ShapeDtypeStruct((_B, _S), jnp.int32),
            jax.ShapeDtypeStruct((_B, _S), jnp.int32),
        ),
        scratch_types=[
            pltpu.VMEM((_BPW, _N1), jnp.int32),
            pltpu.VMEM((_BPW, _N2), jnp.int32),
            pltpu.VMEM((_BPW, _N3), jnp.int32),
            pltpu.VMEM((_BPW, _S), jnp.int32),
            pltpu.VMEM((_BPW, _S), jnp.int32),
            pltpu.VMEM((_BPW, _S), jnp.int32),
            pltpu.VMEM((_BPW, _S), jnp.int32),
            pltpu.VMEM((_BPW, _S), jnp.int32),
        ],
    )
    def body(fps0, fps1, fps2, fps3,
             i04, o1, o2, o3,
             t0, t1, t2, t3, li0, gi1, gi2, gi3):
        wid = lax.axis_index("s") * _NC + lax.axis_index("c")
        b0 = wid * _BPW
        rows = pl.ds(b0, _BPW)
        pltpu.sync_copy(fps0.at[rows], t0)
        pltpu.sync_copy(fps1.at[rows], t1)
        pltpu.sync_copy(fps2.at[rows], t2)
        pltpu.sync_copy(fps3.at[rows], t3)

        for j in range(_BPW):
            jv = jnp.full((16,), j, jnp.int32)

            def chain(i, c, j=j, jv=jv):
                s = pl.ds(i * 16, 16)
                v3 = t3[j, s]
                v2 = plsc.load_gather(t2, [jv, v3])
                v1 = plsc.load_gather(t1, [jv, v2])
                v0 = plsc.load_gather(t0, [jv, v1])
                li0[j, s] = v0
                gi3[j, s] = v3 + (b0 + j) * _N3
                gi2[j, s] = v2 + (b0 + j) * _N2
                gi1[j, s] = v1 + (b0 + j) * _N1
                return c

            lax.fori_loop(0, _S // 16, chain, jnp.int32(0))

        pltpu.sync_copy(li0, i04.at[rows])
        pltpu.sync_copy(gi1, o1.at[rows])
        pltpu.sync_copy(gi2, o2.at[rows])
        pltpu.sync_copy(gi3, o3.at[rows])

    return body


def _sc_gather():
    @functools.partial(
        pl.kernel,
        mesh=_sc_mesh(),
        compiler_params=_SC_PARAMS,
        out_type=(
            jax.ShapeDtypeStruct((_M, _C1), jnp.float32),
            jax.ShapeDtypeStruct((_M, _C2), jnp.float32),
            jax.ShapeDtypeStruct((_M, _C3), jnp.float32),
        ),
        scratch_types=[
            pltpu.VMEM((_BPW, _S), jnp.int32),
            pltpu.VMEM((_BPW, _S), jnp.int32),
            pltpu.VMEM((_BPW, _S), jnp.int32),
            pltpu.VMEM((_CH, _C1), jnp.float32),
            pltpu.VMEM((_CH, _C2), jnp.float32),
            pltpu.VMEM((_CH, _C3), jnp.float32),
            pltpu.VMEM((_CH, _C1), jnp.float32),
            pltpu.VMEM((_CH, _C2), jnp.float32),
            pltpu.VMEM((_CH, _C3), jnp.float32),
            pltpu.SemaphoreType.DMA,
            pltpu.SemaphoreType.DMA,
            pltpu.SemaphoreType.DMA,
            pltpu.SemaphoreType.DMA,
        ],
    )
    def body(o1, o2, o3, f1, f2, f3,
             g1, g2, g3,
             i1b, i2b, i3b, r1a, r2a, r3a, r1b, r2b, r3b,
             gsa, gsb, wsa, wsb):
        wid = lax.axis_index("s") * _NC + lax.axis_index("c")
        b0 = wid * _BPW
        rows = pl.ds(b0, _BPW)
        pltpu.sync_copy(o1.at[rows], i1b)
        pltpu.sync_copy(o2.at[rows], i2b)
        pltpu.sync_copy(o3.at[rows], i3b)

        bufs = ((r1a, r2a, r3a), (r1b, r2b, r3b))
        gsems = (gsa, gsb)
        wsems = (wsa, wsb)
        nch = _BPW * _S // _CH

        def start_g(c):
            s = c % 2
            bj, h = divmod(c * _CH, _S)
            idx = (i1b.at[bj, pl.ds(h, _CH)], i2b.at[bj, pl.ds(h, _CH)],
                   i3b.at[bj, pl.ds(h, _CH)])
            return tuple(
                pltpu.async_copy(f.at[ix], buf, gsems[s])
                for f, ix, buf in zip((f1, f2, f3), idx, bufs[s]))

        def start_w(c):
            s = c % 2
            row = b0 * _S + c * _CH
            return tuple(
                pltpu.async_copy(buf, g.at[pl.ds(row, _CH)], wsems[s])
                for g, buf in zip((g1, g2, g3), bufs[s]))

        pg = {0: start_g(0)}
        pw = {}
        for c in range(1, nch + 1):
            if c <= nch - 1:
                if c >= 2:
                    for w in pw.pop(c - 2):
                        w.wait()
                pg[c] = start_g(c)
            for g in pg.pop(c - 1):
                g.wait()
            pw[c - 1] = start_w(c - 1)
        for w in pw[nch - 2]:
            w.wait()
        for w in pw[nch - 1]:
            w.wait()

    return body


def _mm(x, w):
    return lax.dot_general(x, w, (((1,), (1,)), ((), ())),
                           preferred_element_type=jnp.float32)


def _a0_body(i04b, f0tb, g0o):
    idx = i04b[...]
    f0t = f0tb[...]
    parts = []
    for j in range(_ABLK):
        row = idx[j:j + 1, :]
        lane = jnp.reshape(row % 128, (1, 1, _S))
        chunk = row // 128
        clo = jnp.reshape(jnp.clip(chunk, 0, 7), (1, 1, _S))
        chi = jnp.reshape(jnp.clip(chunk - 8, 0, 7), (1, 1, _S))
        a = jnp.take_along_axis(
            f0t[j], jnp.broadcast_to(lane, (_C0, 16, _S)), axis=2,
            mode="promise_in_bounds")
        lo = jnp.take_along_axis(
            a[:, 0:8, :], jnp.broadcast_to(clo, (_C0, 1, _S)), axis=1,
            mode="promise_in_bounds")
        hi = jnp.take_along_axis(
            a[:, 8:16, :], jnp.broadcast_to(chi, (_C0, 1, _S)), axis=1,
            mode="promise_in_bounds")
        sel = jnp.where(jnp.reshape(chunk < 8, (1, 1, _S)), lo, hi)
        gt = jnp.reshape(sel, (_C0, _S))
        parts.append(jnp.transpose(gt, (1, 0)))
    g0o[...] = jnp.concatenate(parts, axis=0)


def _norm_leaky(z, sm, sq):
    mu = sm * (1.0 / _M)
    ex2 = sq * (1.0 / _M)
    rs = lax.rsqrt(ex2 - mu * mu + _EPS)
    y = (z - mu) * rs
    return jnp.where(y > 0, y, 0.2 * y)


def _dense_body(g0b, g1b, g2b, g3b, f4b,
                w04, w14, w24, w34, b04, b14, b24, b34, w4, b4,
                ob,
                sg0, sg1, sg2, sg3, sf4, sz5, st, st5, stage, sem):
    p = pl.program_id(0)
    i = pl.program_id(1)
    r = pl.ds(i * _BLK, _BLK)

    @pl.when((p == 0) & (i == 0))
    def _init():
        st[...] = jnp.zeros_like(st)
        st5[...] = jnp.zeros_like(st5)

    @pl.when(p == 0)
    def _phase0():
        for k, (gv, sg, w, bb) in enumerate(
                ((g0b[...], sg0, w04, b04), (g1b[...], sg1, w14, b14),
                 (g2b[...], sg2, w24, b24), (g3b[...], sg3, w34, b34))):
            sg[r, :] = gv.astype(jnp.bfloat16)
            z = _mm(gv, w[...]) + bb[...]
            st[k:k + 1, :] += jnp.sum(z, axis=0, keepdims=True)
            st[k + 4:k + 5, :] += jnp.sum(z * z, axis=0, keepdims=True)

    @pl.when(p == 1)
    def _phase1():
        stv = st[...]
        f4v = f4b[...]
        sf4[r, :] = f4v.astype(jnp.bfloat16)
        acc = f4v
        branches = ((sg0[r, :].astype(jnp.float32), w04, b04,
                     stv[0:1, :], stv[4:5, :]),
                    (sg1[r, :].astype(jnp.float32), w14, b14,
                     stv[1:2, :], stv[5:6, :]),
                    (sg2[r, :].astype(jnp.float32), w24, b24,
                     stv[2:3, :], stv[6:7, :]),
                    (sg3[r, :].astype(jnp.float32), w34, b34,
                     stv[3:4, :], stv[7:8, :]))
        for gv, w, bb, sm, sq in branches:
            z = _mm(gv, w[...]) + bb[...]
            acc = acc + _norm_leaky(z, sm, sq)
        z = _mm(acc, w4[...]) + b4[...]
        sz5[r, :] = z.astype(jnp.bfloat16)
        st5[0:1, :] += jnp.sum(z, axis=0, keepdims=True)
        st5[1:2, :] += jnp.sum(z * z, axis=0, keepdims=True)

    @pl.when(p == 2)
    def _phase2():
        stv = st5[...]
        stage[...] = (_norm_leaky(sz5[r, :].astype(jnp.float32),
                                  stv[0:1, :], stv[1:2, :])
                      + sf4[r, :].astype(jnp.float32))
        pltpu.async_copy(stage, ob.at[r, :], sem).wait()


def _rows(c, blk=_BLK):
    return pl.BlockSpec((blk, c), lambda i: (i, 0))


def _rep(shape):
    return pl.BlockSpec(shape, lambda i: tuple(0 for _ in shape))


_ARB = pltpu.CompilerParams(dimension_semantics=("arbitrary",))


def kernel(num_point, f0, f1, f2, f3, f4, FPS_0, FPS_1, FPS_2, FPS_3,
           W04, b04, W14, b14, W24, b24, W34, b34, W4, b4):
    i04, o1, o2, o3 = _sc_chain()(FPS_0, FPS_1, FPS_2, FPS_3)
    g1, g2, g3 = _sc_gather()(
        o1, o2, o3,
        f1.reshape(_B * _N1, _C1),
        f2.reshape(_B * _N2, _C2), f3.reshape(_B * _N3, _C3))

    b04r, b14r = b04.reshape(1, _C4), b14.reshape(1, _C4)
    b24r, b34r = b24.reshape(1, _C4), b34.reshape(1, _C4)
    b4r = b4.reshape(1, _C4)
    f4f = f4.reshape(_M, _C4)
    f0t = jnp.transpose(f0, (0, 2, 1)).reshape(_B, _C0, 16, 128)
    arows = _ABLK * _S

    g0 = pl.pallas_call(
        _a0_body,
        grid=(_B // _ABLK,),
        in_specs=[_rows(_S, _ABLK),
                  pl.BlockSpec((_ABLK, _C0, 16, 128),
                               lambda i: (i, 0, 0, 0))],
        out_specs=_rows(_C0, arows),
        out_shape=jax.ShapeDtypeStruct((_M, _C0), jnp.float32),
        compiler_params=_ARB,
    )(i04, f0t)

    def _prow(c, cond):
        return pl.BlockSpec(
            (_BLK, c), lambda p, i: (jnp.where(cond(p), i, 0), 0))

    def _prep(shape):
        return pl.BlockSpec(shape, lambda p, i: tuple(0 for _ in shape))

    out = pl.pallas_call(
        _dense_body,
        grid=(3, _M // _BLK),
        in_specs=[_prow(_C0, lambda p: p == 0),
                  _prow(_C1, lambda p: p == 0),
                  _prow(_C2, lambda p: p == 0),
                  _prow(_C3, lambda p: p == 0),
                  _prow(_C4, lambda p: p == 1),
                  _prep((_C4, _C0)), _prep((_C4, _C1)),
                  _prep((_C4, _C2)), _prep((_C4, _C3)),
                  _prep((1, _C4)), _prep((1, _C4)),
                  _prep((1, _C4)), _prep((1, _C4)),
                  _prep((_C4, _C4)), _prep((1, _C4))],
        out_specs=pl.BlockSpec(memory_space=pl.ANY),
        out_shape=jax.ShapeDtypeStruct((_M, _C4), jnp.float32),
        scratch_shapes=[
            pltpu.VMEM((_M, _C0), jnp.bfloat16),
            pltpu.VMEM((_M, _C1), jnp.bfloat16),
            pltpu.VMEM((_M, _C2), jnp.bfloat16),
            pltpu.VMEM((_M, _C3), jnp.bfloat16),
            pltpu.VMEM((_M, _C4), jnp.bfloat16),
            pltpu.VMEM((_M, _C4), jnp.bfloat16),
            pltpu.VMEM((8, _C4), jnp.float32),
            pltpu.VMEM((2, _C4), jnp.float32),
            pltpu.VMEM((_BLK, _C4), jnp.float32),
            pltpu.SemaphoreType.DMA,
        ],
        compiler_params=pltpu.CompilerParams(
            dimension_semantics=("arbitrary", "arbitrary"),
            vmem_limit_bytes=120 * 1024 * 1024),
    )(g0, g1, g2, g3, f4f, W04, W14, W24, W34,
      b04r, b14r, b24r, b34r, W4, b4r)

    return out.reshape(_B, _S, _C4)


# speedup vs baseline: 1.3869x; 1.3869x over previous
import functools

import jax
import jax.numpy as jnp
from jax import lax
from jax.experimental import pallas as pl
from jax.experimental.pallas import tpu as pltpu
from jax.experimental.pallas import tpu_sc as plsc

_B = 128
_S = 128
_N0, _N1, _N2, _N3 = 2048, 1024, 512, 256
_C0, _C1, _C2, _C3, _C4 = 64, 128, 128, 256, 256
_M = _B * _S
_EPS = 1e-5
_NC, _NS = 2, 16
_NW = _NC * _NS
_BPW = _B // _NW
_ABLK = 8
_BLK = 2048
_CH = 64

_SC_PARAMS = pltpu.CompilerParams(needs_layout_passes=False)


def _sc_mesh():
    return plsc.VectorSubcoreMesh(
        core_axis_name="c", subcore_axis_name="s",
        num_cores=_NC, num_subcores=_NS)


def _sc_chain():
    @functools.partial(
        pl.kernel,
        mesh=_sc_mesh(),
        compiler_params=_SC_PARAMS,
        out_type=(
            jax.ShapeDtypeStruct((_B, _S), jnp.int32),
            jax.ShapeDtypeStruct((_B, _S), jnp.int32),
            jax.ShapeDtypeStruct((_B, _S), jnp.int32),
            jax.ShapeDtypeStruct((_B, _S), jnp.int32),
        ),
        scratch_types=[
            pltpu.VMEM((_BPW, _N1), jnp.int32),
            pltpu.VMEM((_BPW, _N2), jnp.int32),
            pltpu.VMEM((_BPW, _N3), jnp.int32),
            pltpu.VMEM((_BPW, _S), jnp.int32),
            pltpu.VMEM((_BPW, _S), jnp.int32),
            pltpu.VMEM((_BPW, _S), jnp.int32),
            pltpu.VMEM((_BPW, _S), jnp.int32),
            pltpu.VMEM((_BPW, _S), jnp.int32),
        ],
    )
    def body(fps0, fps1, fps2, fps3,
             i04, o1, o2, o3,
             t0, t1, t2, t3, li0, gi1, gi2, gi3):
        wid = lax.axis_index("s") * _NC + lax.axis_index("c")
        b0 = wid * _BPW
        rows = pl.ds(b0, _BPW)
        pltpu.sync_copy(fps0.at[rows], t0)
        pltpu.sync_copy(fps1.at[rows], t1)
        pltpu.sync_copy(fps2.at[rows], t2)
        pltpu.sync_copy(fps3.at[rows], t3)

        for j in range(_BPW):
            jv = jnp.full((16,), j, jnp.int32)

            def chain(i, c, j=j, jv=jv):
                s = pl.ds(i * 16, 16)
                v3 = t3[j, s]
                v2 = plsc.load_gather(t2, [jv, v3])
                v1 = plsc.load_gather(t1, [jv, v2])
                v0 = plsc.load_gather(t0, [jv, v1])
                li0[j, s] = v0
                gi3[j, s] = v3 + (b0 + j) * _N3
                gi2[j, s] = v2 + (b0 + j) * _N2
                gi1[j, s] = v1 + (b0 + j) * _N1
                return c

            lax.fori_loop(0, _S // 16, chain, jnp.int32(0))

        pltpu.sync_copy(li0, i04.at[rows])
        pltpu.sync_copy(gi1, o1.at[rows])
        pltpu.sync_copy(gi2, o2.at[rows])
        pltpu.sync_copy(gi3, o3.at[rows])

    return body


def _sc_gather():
    @functools.partial(
        pl.kernel,
        mesh=_sc_mesh(),
        compiler_params=_SC_PARAMS,
        out_type=(
            jax.ShapeDtypeStruct((_M, _C1), jnp.float32),
            jax.ShapeDtypeStruct((_M, _C2), jnp.float32),
            jax.ShapeDtypeStruct((_M, _C3), jnp.float32),
        ),
        scratch_types=[
            pltpu.VMEM((_BPW, _S), jnp.int32),
            pltpu.VMEM((_BPW, _S), jnp.int32),
            pltpu.VMEM((_BPW, _S), jnp.int32),
            pltpu.VMEM((_CH, _C1), jnp.float32),
            pltpu.VMEM((_CH, _C2), jnp.float32),
            pltpu.VMEM((_CH, _C3), jnp.float32),
            pltpu.VMEM((_CH, _C1), jnp.float32),
            pltpu.VMEM((_CH, _C2), jnp.float32),
            pltpu.VMEM((_CH, _C3), jnp.float32),
            pltpu.SemaphoreType.DMA,
            pltpu.SemaphoreType.DMA,
            pltpu.SemaphoreType.DMA,
            pltpu.SemaphoreType.DMA,
        ],
    )
    def body(o1, o2, o3, f1, f2, f3,
             g1, g2, g3,
             i1b, i2b, i3b, r1a, r2a, r3a, r1b, r2b, r3b,
             gsa, gsb, wsa, wsb):
        wid = lax.axis_index("s") * _NC + lax.axis_index("c")
        b0 = wid * _BPW
        rows = pl.ds(b0, _BPW)
        pltpu.sync_copy(o1.at[rows], i1b)
        pltpu.sync_copy(o2.at[rows], i2b)
        pltpu.sync_copy(o3.at[rows], i3b)

        bufs = ((r1a, r2a, r3a), (r1b, r2b, r3b))
        gsems = (gsa, gsb)
        wsems = (wsa, wsb)
        nch = _BPW * _S // _CH

        def start_g(c):
            s = c % 2
            bj, h = divmod(c * _CH, _S)
            idx = (i1b.at[bj, pl.ds(h, _CH)], i2b.at[bj, pl.ds(h, _CH)],
                   i3b.at[bj, pl.ds(h, _CH)])
            return tuple(
                pltpu.async_copy(f.at[ix], buf, gsems[s])
                for f, ix, buf in zip((f1, f2, f3), idx, bufs[s]))

        def start_w(c):
            s = c % 2
            row = b0 * _S + c * _CH
            return tuple(
                pltpu.async_copy(buf, g.at[pl.ds(row, _CH)], wsems[s])
                for g, buf in zip((g1, g2, g3), bufs[s]))

        pg = {0: start_g(0)}
        pw = {}
        for c in range(1, nch + 1):
            if c <= nch - 1:
                if c >= 2:
                    for w in pw.pop(c - 2):
                        w.wait()
                pg[c] = start_g(c)
            for g in pg.pop(c - 1):
                g.wait()
            pw[c - 1] = start_w(c - 1)
        for w in pw[nch - 2]:
            w.wait()
        for w in pw[nch - 1]:
            w.wait()

    return body


def _mm(x, w):
    return lax.dot_general(x, w, (((1,), (1,)), ((), ())),
                           preferred_element_type=jnp.float32)


def _a0_body(i04b, f0tb, w04, b04, g0o, st0):
    i = pl.program_id(0)

    @pl.when(i == 0)
    def _init():
        st0[...] = jnp.zeros_like(st0)

    idx = i04b[...]
    f0t = f0tb[...].astype(jnp.bfloat16)
    parts = []
    for j in range(_ABLK):
        row = idx[j:j + 1, :]
        n_iota = lax.broadcasted_iota(jnp.int32, (_N0, _S), 0)
        oh = (n_iota == row).astype(jnp.bfloat16)
        parts.append(lax.dot_general(
            oh, f0t[j], (((0,), (1,)), ((), ())),
            preferred_element_type=jnp.float32))
    g0 = jnp.concatenate(parts, axis=0)
    g0o[...] = g0
    z = _mm(g0, w04[...]) + b04[...]
    st0[0:1, :] += jnp.sum(z, axis=0, keepdims=True)
    st0[1:2, :] += jnp.sum(z * z, axis=0, keepdims=True)


def _norm_leaky(z, sm, sq):
    mu = sm * (1.0 / _M)
    ex2 = sq * (1.0 / _M)
    rs = lax.rsqrt(ex2 - mu * mu + _EPS)
    y = (z - mu) * rs
    return jnp.where(y > 0, y, 0.2 * y)


def _dense_body(st0, g0b, g1b, g2b, g3b, f4b,
                w04, w14, w24, w34, b04, b14, b24, b34, w4, b4,
                ob,
                sg1, sg2, sg3, sf4, sz5, st, st5, stage, sem):
    p = pl.program_id(0)
    i = pl.program_id(1)
    r = pl.ds(i * _BLK, _BLK)

    @pl.when((p == 0) & (i == 0))
    def _init():
        st[...] = jnp.zeros_like(st)
        st5[...] = jnp.zeros_like(st5)

    @pl.when(p == 0)
    def _phase0():
        for k, (gv, sg, w, bb) in enumerate(
                ((g1b[...], sg1, w14, b14),
                 (g2b[...], sg2, w24, b24), (g3b[...], sg3, w34, b34))):
            sg[r, :] = gv.astype(jnp.bfloat16)
            z = _mm(gv, w[...]) + bb[...]
            st[k:k + 1, :] += jnp.sum(z, axis=0, keepdims=True)
            st[k + 3:k + 4, :] += jnp.sum(z * z, axis=0, keepdims=True)

    @pl.when(p == 1)
    def _phase1():
        st0v = st0[...]
        stv = st[...]
        f4v = f4b[...]
        sf4[r, :] = f4v.astype(jnp.bfloat16)
        acc = f4v
        branches = ((g0b[...], w04, b04,
                     st0v[0:1, :], st0v[1:2, :]),
                    (sg1[r, :].astype(jnp.float32), w14, b14,
                     stv[0:1, :], stv[3:4, :]),
                    (sg2[r, :].astype(jnp.float32), w24, b24,
                     stv[1:2, :], stv[4:5, :]),
                    (sg3[r, :].astype(jnp.float32), w34, b34,
                     stv[2:3, :], stv[5:6, :]))
        for gv, w, bb, sm, sq in branches:
            z = _mm(gv, w[...]) + bb[...]
            acc = acc + _norm_leaky(z, sm, sq)
        z = _mm(acc, w4[...]) + b4[...]
        sz5[r, :] = z.astype(jnp.bfloat16)
        st5[0:1, :] += jnp.sum(z, axis=0, keepdims=True)
        st5[1:2, :] += jnp.sum(z * z, axis=0, keepdims=True)

    @pl.when(p == 2)
    def _phase2():
        stv = st5[...]
        stage[...] = (_norm_leaky(sz5[r, :].astype(jnp.float32),
                                  stv[0:1, :], stv[1:2, :])
                      + sf4[r, :].astype(jnp.float32))
        pltpu.async_copy(stage, ob.at[r, :], sem).wait()


def _rows(c, blk=_BLK):
    return pl.BlockSpec((blk, c), lambda i: (i, 0))


def _rep(shape):
    return pl.BlockSpec(shape, lambda i: tuple(0 for _ in shape))


_ARB = pltpu.CompilerParams(dimension_semantics=("arbitrary",))


def kernel(num_point, f0, f1, f2, f3, f4, FPS_0, FPS_1, FPS_2, FPS_3,
           W04, b04, W14, b14, W24, b24, W34, b34, W4, b4):
    i04, o1, o2, o3 = _sc_chain()(FPS_0, FPS_1, FPS_2, FPS_3)
    g1, g2, g3 = _sc_gather()(
        o1, o2, o3,
        f1.reshape(_B * _N1, _C1),
        f2.reshape(_B * _N2, _C2), f3.reshape(_B * _N3, _C3))

    b04r, b14r = b04.reshape(1, _C4), b14.reshape(1, _C4)
    b24r, b34r = b24.reshape(1, _C4), b34.reshape(1, _C4)
    b4r = b4.reshape(1, _C4)
    f4f = f4.reshape(_M, _C4)
    f0t = jnp.transpose(f0, (0, 2, 1))
    arows = _ABLK * _S

    g0, st0 = pl.pallas_call(
        _a0_body,
        grid=(_B // _ABLK,),
        in_specs=[_rows(_S, _ABLK),
                  pl.BlockSpec((_ABLK, _C0, _N0), lambda i: (i, 0, 0)),
                  _rep((_C4, _C0)), _rep((1, _C4))],
        out_specs=(_rows(_C0, arows), _rep((2, _C4))),
        out_shape=(jax.ShapeDtypeStruct((_M, _C0), jnp.float32),
                   jax.ShapeDtypeStruct((2, _C4), jnp.float32)),
        compiler_params=_ARB,
    )(i04, f0t, W04, b04r)

    def _prow(c, cond):
        return pl.BlockSpec(
            (_BLK, c), lambda p, i: (jnp.where(cond(p), i, 0), 0))

    def _prep(shape):
        return pl.BlockSpec(shape, lambda p, i: tuple(0 for _ in shape))

    out = pl.pallas_call(
        _dense_body,
        grid=(3, _M // _BLK),
        in_specs=[_prep((2, _C4)),
                  _prow(_C0, lambda p: p == 1),
                  _prow(_C1, lambda p: p == 0),
                  _prow(_C2, lambda p: p == 0),
                  _prow(_C3, lambda p: p == 0),
                  _prow(_C4, lambda p: p == 1),
                  _prep((_C4, _C0)), _prep((_C4, _C1)),
                  _prep((_C4, _C2)), _prep((_C4, _C3)),
                  _prep((1, _C4)), _prep((1, _C4)),
                  _prep((1, _C4)), _prep((1, _C4)),
                  _prep((_C4, _C4)), _prep((1, _C4))],
        out_specs=pl.BlockSpec(memory_space=pl.ANY),
        out_shape=jax.ShapeDtypeStruct((_M, _C4), jnp.float32),
        scratch_shapes=[
            pltpu.VMEM((_M, _C1), jnp.bfloat16),
            pltpu.VMEM((_M, _C2), jnp.bfloat16),
            pltpu.VMEM((_M, _C3), jnp.bfloat16),
            pltpu.VMEM((_M, _C4), jnp.bfloat16),
            pltpu.VMEM((_M, _C4), jnp.bfloat16),
            pltpu.VMEM((6, _C4), jnp.float32),
            pltpu.VMEM((2, _C4), jnp.float32),
            pltpu.VMEM((_BLK, _C4), jnp.float32),
            pltpu.SemaphoreType.DMA,
        ],
        compiler_params=pltpu.CompilerParams(
            dimension_semantics=("arbitrary", "arbitrary"),
            vmem_limit_bytes=120 * 1024 * 1024),
    )(st0, g0, g1, g2, g3, f4f, W04, W14, W24, W34,
      b04r, b14r, b24r, b34r, W4, b4r)

    return out.reshape(_B, _S, _C4)


# speedup vs baseline: 1.3902x; 1.0023x over previous
import functools

import jax
import jax.numpy as jnp
from jax import lax
from jax.experimental import pallas as pl
from jax.experimental.pallas import tpu as pltpu
from jax.experimental.pallas import tpu_sc as plsc

_B = 128
_S = 128
_N0, _N1, _N2, _N3 = 2048, 1024, 512, 256
_C0, _C1, _C2, _C3, _C4 = 64, 128, 128, 256, 256
_M = _B * _S
_EPS = 1e-5
_NC, _NS = 2, 16
_NW = _NC * _NS
_BPW = _B // _NW
_ABLK = 8
_BLK = 2048
_CH = 64

_SC_PARAMS = pltpu.CompilerParams(needs_layout_passes=False)


def _sc_mesh():
    return plsc.VectorSubcoreMesh(
        core_axis_name="c", subcore_axis_name="s",
        num_cores=_NC, num_subcores=_NS)


def _sc_chain():
    @functools.partial(
        pl.kernel,
        mesh=_sc_mesh(),
        compiler_params=_SC_PARAMS,
        out_type=(
            jax.ShapeDtypeStruct((_B, _S), jnp.int32),
            jax.ShapeDtypeStruct((_B, _S), jnp.int32),
            jax.ShapeDtypeStruct((_B, _S), jnp.int32),
            jax.ShapeDtypeStruct((_B, _S), jnp.int32),
        ),
        scratch_types=[
            pltpu.VMEM((_BPW, _N1), jnp.int32),
            pltpu.VMEM((_BPW, _N2), jnp.int32),
            pltpu.VMEM((_BPW, _N3), jnp.int32),
            pltpu.VMEM((_BPW, _S), jnp.int32),
            pltpu.VMEM((_BPW, _S), jnp.int32),
            pltpu.VMEM((_BPW, _S), jnp.int32),
            pltpu.VMEM((_BPW, _S), jnp.int32),
            pltpu.VMEM((_BPW, _S), jnp.int32),
        ],
    )
    def body(fps0, fps1, fps2, fps3,
             i04, o1, o2, o3,
             t0, t1, t2, t3, li0, gi1, gi2, gi3):
        wid = lax.axis_index("s") * _NC + lax.axis_index("c")
        b0 = wid * _BPW
        rows = pl.ds(b0, _BPW)
        pltpu.sync_copy(fps0.at[rows], t0)
        pltpu.sync_copy(fps1.at[rows], t1)
        pltpu.sync_copy(fps2.at[rows], t2)
        pltpu.sync_copy(fps3.at[rows], t3)

        for j in range(_BPW):
            jv = jnp.full((16,), j, jnp.int32)

            def chain(i, c, j=j, jv=jv):
                s = pl.ds(i * 16, 16)
                v3 = t3[j, s]
                v2 = plsc.load_gather(t2, [jv, v3])
                v1 = plsc.load_gather(t1, [jv, v2])
                v0 = plsc.load_gather(t0, [jv, v1])
                li0[j, s] = v0
                gi3[j, s] = v3 + (b0 + j) * _N3
                gi2[j, s] = v2 + (b0 + j) * _N2
                gi1[j, s] = v1 + (b0 + j) * _N1
                return c

            lax.fori_loop(0, _S // 16, chain, jnp.int32(0))

        pltpu.sync_copy(li0, i04.at[rows])
        pltpu.sync_copy(gi1, o1.at[rows])
        pltpu.sync_copy(gi2, o2.at[rows])
        pltpu.sync_copy(gi3, o3.at[rows])

    return body


def _sc_gather():
    @functools.partial(
        pl.kernel,
        mesh=_sc_mesh(),
        compiler_params=_SC_PARAMS,
        out_type=(
            jax.ShapeDtypeStruct((_M, _C1), jnp.float32),
            jax.ShapeDtypeStruct((_M, _C2), jnp.float32),
            jax.ShapeDtypeStruct((_M, _C3), jnp.float32),
        ),
        scratch_types=[
            pltpu.VMEM((_BPW, _S), jnp.int32),
            pltpu.VMEM((_BPW, _S), jnp.int32),
            pltpu.VMEM((_BPW, _S), jnp.int32),
            pltpu.VMEM((_CH, _C1), jnp.float32),
            pltpu.VMEM((_CH, _C2), jnp.float32),
            pltpu.VMEM((_CH, _C3), jnp.float32),
            pltpu.VMEM((_CH, _C1), jnp.float32),
            pltpu.VMEM((_CH, _C2), jnp.float32),
            pltpu.VMEM((_CH, _C3), jnp.float32),
            pltpu.SemaphoreType.DMA,
            pltpu.SemaphoreType.DMA,
            pltpu.SemaphoreType.DMA,
            pltpu.SemaphoreType.DMA,
        ],
    )
    def body(o1, o2, o3, f1, f2, f3,
             g1, g2, g3,
             i1b, i2b, i3b, r1a, r2a, r3a, r1b, r2b, r3b,
             gsa, gsb, wsa, wsb):
        wid = lax.axis_index("s") * _NC + lax.axis_index("c")
        b0 = wid * _BPW
        rows = pl.ds(b0, _BPW)
        pltpu.sync_copy(o1.at[rows], i1b)
        pltpu.sync_copy(o2.at[rows], i2b)
        pltpu.sync_copy(o3.at[rows], i3b)

        bufs = ((r1a, r2a, r3a), (r1b, r2b, r3b))
        gsems = (gsa, gsb)
        wsems = (wsa, wsb)
        nch = _BPW * _S // _CH

        def start_g(c):
            s = c % 2
            bj, h = divmod(c * _CH, _S)
            idx = (i1b.at[bj, pl.ds(h, _CH)], i2b.at[bj, pl.ds(h, _CH)],
                   i3b.at[bj, pl.ds(h, _CH)])
            return tuple(
                pltpu.async_copy(f.at[ix], buf, gsems[s])
                for f, ix, buf in zip((f1, f2, f3), idx, bufs[s]))

        def start_w(c):
            s = c % 2
            row = b0 * _S + c * _CH
            return tuple(
                pltpu.async_copy(buf, g.at[pl.ds(row, _CH)], wsems[s])
                for g, buf in zip((g1, g2, g3), bufs[s]))

        pg = {0: start_g(0)}
        pw = {}
        for c in range(1, nch + 1):
            if c <= nch - 1:
                if c >= 2:
                    for w in pw.pop(c - 2):
                        w.wait()
                pg[c] = start_g(c)
            for g in pg.pop(c - 1):
                g.wait()
            pw[c - 1] = start_w(c - 1)
        for w in pw[nch - 2]:
            w.wait()
        for w in pw[nch - 1]:
            w.wait()

    return body


def _mm(x, w):
    return lax.dot_general(x, w, (((1,), (1,)), ((), ())),
                           preferred_element_type=jnp.float32)


def _a0_body(i04b, f0tb, w04, b04, g0o, st0):
    i = pl.program_id(0)

    @pl.when(i == 0)
    def _init():
        st0[...] = jnp.zeros_like(st0)

    idx = i04b[...]
    f0t = f0tb[...].astype(jnp.bfloat16)
    parts = []
    for j in range(_ABLK):
        row = idx[j:j + 1, :]
        n_iota = lax.broadcasted_iota(jnp.int32, (_N0, _S), 0)
        oh = (n_iota == row).astype(jnp.bfloat16)
        parts.append(lax.dot_general(
            oh, f0t[j], (((0,), (1,)), ((), ())),
            preferred_element_type=jnp.float32))
    g0 = jnp.concatenate(parts, axis=0)
    g0o[...] = g0
    z = _mm(g0, w04[...]) + b04[...]
    st0[0:1, :] += jnp.sum(z, axis=0, keepdims=True)
    st0[1:2, :] += jnp.sum(z * z, axis=0, keepdims=True)


def _norm_leaky(z, sm, sq):
    mu = sm * (1.0 / _M)
    ex2 = sq * (1.0 / _M)
    rs = lax.rsqrt(ex2 - mu * mu + _EPS)
    y = (z - mu) * rs
    return jnp.where(y > 0, y, 0.2 * y)


def _dense_body(st0, g0b, g1b, g2b, g3b, f4b,
                w04, w14, w24, w34, b04, b14, b24, b34, w4, b4,
                ob,
                sg1, sg2, sg3, sf4, sz5, st, st5, stage, sem):
    p = pl.program_id(0)
    i = pl.program_id(1)
    r = pl.ds(i * _BLK, _BLK)

    @pl.when((p == 0) & (i == 0))
    def _init():
        st[...] = jnp.zeros_like(st)
        st5[...] = jnp.zeros_like(st5)

    @pl.when(p == 0)
    def _phase0():
        for k, (gv, sg, w, bb) in enumerate(
                ((g1b[...], sg1, w14, b14),
                 (g2b[...], sg2, w24, b24), (g3b[...], sg3, w34, b34))):
            sg[r, :] = gv.astype(jnp.bfloat16)
            z = _mm(gv, w[...]) + bb[...]
            st[k:k + 1, :] += jnp.sum(z, axis=0, keepdims=True)
            st[k + 3:k + 4, :] += jnp.sum(z * z, axis=0, keepdims=True)

    @pl.when(p == 1)
    def _phase1():
        st0v = st0[...]
        stv = st[...]
        f4v = f4b[...]
        sf4[r, :] = f4v.astype(jnp.bfloat16)
        acc = f4v
        branches = ((g0b[...], w04, b04,
                     st0v[0:1, :], st0v[1:2, :]),
                    (sg1[r, :].astype(jnp.float32), w14, b14,
                     stv[0:1, :], stv[3:4, :]),
                    (sg2[r, :].astype(jnp.float32), w24, b24,
                     stv[1:2, :], stv[4:5, :]),
                    (sg3[r, :].astype(jnp.float32), w34, b34,
                     stv[2:3, :], stv[5:6, :]))
        for gv, w, bb, sm, sq in branches:
            z = _mm(gv, w[...]) + bb[...]
            acc = acc + _norm_leaky(z, sm, sq)
        z = _mm(acc, w4[...]) + b4[...]
        sz5[r, :] = z.astype(jnp.bfloat16)
        st5[0:1, :] += jnp.sum(z, axis=0, keepdims=True)
        st5[1:2, :] += jnp.sum(z * z, axis=0, keepdims=True)

    @pl.when(p == 2)
    def _phase2():
        stv = st5[...]
        stage[...] = (_norm_leaky(sz5[r, :].astype(jnp.float32),
                                  stv[0:1, :], stv[1:2, :])
                      + sf4[r, :].astype(jnp.float32))
        pltpu.async_copy(stage, ob.at[r, :], sem).wait()


def _rows(c, blk=_BLK):
    return pl.BlockSpec((blk, c), lambda i: (i, 0))


def _rep(shape):
    return pl.BlockSpec(shape, lambda i: tuple(0 for _ in shape))


_ARB = pltpu.CompilerParams(dimension_semantics=("arbitrary",))


def kernel(num_point, f0, f1, f2, f3, f4, FPS_0, FPS_1, FPS_2, FPS_3,
           W04, b04, W14, b14, W24, b24, W34, b34, W4, b4):
    i04, o1, o2, o3 = _sc_chain()(FPS_0, FPS_1, FPS_2, FPS_3)
    g1, g2, g3 = _sc_gather()(
        o1, o2, o3,
        f1.reshape(_B * _N1, _C1),
        f2.reshape(_B * _N2, _C2), f3.reshape(_B * _N3, _C3))

    b04r, b14r = b04.reshape(1, _C4), b14.reshape(1, _C4)
    b24r, b34r = b24.reshape(1, _C4), b34.reshape(1, _C4)
    b4r = b4.reshape(1, _C4)
    f4f = f4.reshape(_M, _C4)
    f0t = jnp.transpose(f0, (0, 2, 1))
    arows = _ABLK * _S

    g0, st0 = pl.pallas_call(
        _a0_body,
        grid=(_B // _ABLK,),
        in_specs=[_rows(_S, _ABLK),
                  pl.BlockSpec((_ABLK, _C0, _N0), lambda i: (i, 0, 0)),
                  _rep((_C4, _C0)), _rep((1, _C4))],
        out_specs=(_rows(_C0, arows), _rep((2, _C4))),
        out_shape=(jax.ShapeDtypeStruct((_M, _C0), jnp.float32),
                   jax.ShapeDtypeStruct((2, _C4), jnp.float32)),
        compiler_params=pltpu.CompilerParams(
            dimension_semantics=("arbitrary",),
            vmem_limit_bytes=100 * 1024 * 1024),
    )(i04, f0t, W04, b04r)

    def _prow(c, cond):
        return pl.BlockSpec(
            (_BLK, c), lambda p, i: (jnp.where(cond(p), i, 0), 0))

    def _prep(shape):
        return pl.BlockSpec(shape, lambda p, i: tuple(0 for _ in shape))

    out = pl.pallas_call(
        _dense_body,
        grid=(3, _M // _BLK),
        in_specs=[_prep((2, _C4)),
                  _prow(_C0, lambda p: p == 1),
                  _prow(_C1, lambda p: p == 0),
                  _prow(_C2, lambda p: p == 0),
                  _prow(_C3, lambda p: p == 0),
                  _prow(_C4, lambda p: p == 1),
                  _prep((_C4, _C0)), _prep((_C4, _C1)),
                  _prep((_C4, _C2)), _prep((_C4, _C3)),
                  _prep((1, _C4)), _prep((1, _C4)),
                  _prep((1, _C4)), _prep((1, _C4)),
                  _prep((_C4, _C4)), _prep((1, _C4))],
        out_specs=pl.BlockSpec(memory_space=pl.ANY),
        out_shape=jax.ShapeDtypeStruct((_M, _C4), jnp.float32),
        scratch_shapes=[
            pltpu.VMEM((_M, _C1), jnp.bfloat16),
            pltpu.VMEM((_M, _C2), jnp.bfloat16),
            pltpu.VMEM((_M, _C3), jnp.bfloat16),
            pltpu.VMEM((_M, _C4), jnp.bfloat16),
            pltpu.VMEM((_M, _C4), jnp.bfloat16),
            pltpu.VMEM((6, _C4), jnp.float32),
            pltpu.VMEM((2, _C4), jnp.float32),
            pltpu.VMEM((_BLK, _C4), jnp.float32),
            pltpu.SemaphoreType.DMA,
        ],
        compiler_params=pltpu.CompilerParams(
            dimension_semantics=("arbitrary", "arbitrary"),
            vmem_limit_bytes=120 * 1024 * 1024),
    )(st0, g0, g1, g2, g3, f4f, W04, W14, W24, W34,
      b04r, b14r, b24r, b34r, W4, b4r)

    return out.reshape(_B, _S, _C4)


# speedup vs baseline: 1.4043x; 1.0101x over previous
import functools

import jax
import jax.numpy as jnp
from jax import lax
from jax.experimental import pallas as pl
from jax.experimental.pallas import tpu as pltpu
from jax.experimental.pallas import tpu_sc as plsc

_B = 128
_S = 128
_N0, _N1, _N2, _N3 = 2048, 1024, 512, 256
_C0, _C1, _C2, _C3, _C4 = 64, 128, 128, 256, 256
_M = _B * _S
_EPS = 1e-5
_NC, _NS = 2, 16
_NW = _NC * _NS
_BPW = _B // _NW
_ABLK = 16
_BLK = 2048
_CH = 64

_SC_PARAMS = pltpu.CompilerParams(needs_layout_passes=False)


def _sc_mesh():
    return plsc.VectorSubcoreMesh(
        core_axis_name="c", subcore_axis_name="s",
        num_cores=_NC, num_subcores=_NS)


def _sc_chain():
    @functools.partial(
        pl.kernel,
        mesh=_sc_mesh(),
        compiler_params=_SC_PARAMS,
        out_type=(
            jax.ShapeDtypeStruct((_B, _S), jnp.int32),
            jax.ShapeDtypeStruct((_B, _S), jnp.int32),
            jax.ShapeDtypeStruct((_B, _S), jnp.int32),
            jax.ShapeDtypeStruct((_B, _S), jnp.int32),
        ),
        scratch_types=[
            pltpu.VMEM((_BPW, _N1), jnp.int32),
            pltpu.VMEM((_BPW, _N2), jnp.int32),
            pltpu.VMEM((_BPW, _N3), jnp.int32),
            pltpu.VMEM((_BPW, _S), jnp.int32),
            pltpu.VMEM((_BPW, _S), jnp.int32),
            pltpu.VMEM((_BPW, _S), jnp.int32),
            pltpu.VMEM((_BPW, _S), jnp.int32),
            pltpu.VMEM((_BPW, _S), jnp.int32),
        ],
    )
    def body(fps0, fps1, fps2, fps3,
             i04, o1, o2, o3,
             t0, t1, t2, t3, li0, gi1, gi2, gi3):
        wid = lax.axis_index("s") * _NC + lax.axis_index("c")
        b0 = wid * _BPW
        rows = pl.ds(b0, _BPW)
        pltpu.sync_copy(fps0.at[rows], t0)
        pltpu.sync_copy(fps1.at[rows], t1)
        pltpu.sync_copy(fps2.at[rows], t2)
        pltpu.sync_copy(fps3.at[rows], t3)

        for j in range(_BPW):
            jv = jnp.full((16,), j, jnp.int32)

            def chain(i, c, j=j, jv=jv):
                s = pl.ds(i * 16, 16)
                v3 = t3[j, s]
                v2 = plsc.load_gather(t2, [jv, v3])
                v1 = plsc.load_gather(t1, [jv, v2])
                v0 = plsc.load_gather(t0, [jv, v1])
                li0[j, s] = v0
                gi3[j, s] = v3 + (b0 + j) * _N3
                gi2[j, s] = v2 + (b0 + j) * _N2
                gi1[j, s] = v1 + (b0 + j) * _N1
                return c

            lax.fori_loop(0, _S // 16, chain, jnp.int32(0))

        pltpu.sync_copy(li0, i04.at[rows])
        pltpu.sync_copy(gi1, o1.at[rows])
        pltpu.sync_copy(gi2, o2.at[rows])
        pltpu.sync_copy(gi3, o3.at[rows])

    return body


def _sc_gather():
    @functools.partial(
        pl.kernel,
        mesh=_sc_mesh(),
        compiler_params=_SC_PARAMS,
        out_type=(
            jax.ShapeDtypeStruct((_M, _C1), jnp.float32),
            jax.ShapeDtypeStruct((_M, _C2), jnp.float32),
            jax.ShapeDtypeStruct((_M, _C3), jnp.float32),
        ),
        scratch_types=[
            pltpu.VMEM((_BPW, _S), jnp.int32),
            pltpu.VMEM((_BPW, _S), jnp.int32),
            pltpu.VMEM((_BPW, _S), jnp.int32),
            pltpu.VMEM((_CH, _C1), jnp.float32),
            pltpu.VMEM((_CH, _C2), jnp.float32),
            pltpu.VMEM((_CH, _C3), jnp.float32),
            pltpu.VMEM((_CH, _C1), jnp.float32),
            pltpu.VMEM((_CH, _C2), jnp.float32),
            pltpu.VMEM((_CH, _C3), jnp.float32),
            pltpu.SemaphoreType.DMA,
            pltpu.SemaphoreType.DMA,
            pltpu.SemaphoreType.DMA,
            pltpu.SemaphoreType.DMA,
        ],
    )
    def body(o1, o2, o3, f1, f2, f3,
             g1, g2, g3,
             i1b, i2b, i3b, r1a, r2a, r3a, r1b, r2b, r3b,
             gsa, gsb, wsa, wsb):
        wid = lax.axis_index("s") * _NC + lax.axis_index("c")
        b0 = wid * _BPW
        rows = pl.ds(b0, _BPW)
        pltpu.sync_copy(o1.at[rows], i1b)
        pltpu.sync_copy(o2.at[rows], i2b)
        pltpu.sync_copy(o3.at[rows], i3b)

        bufs = ((r1a, r2a, r3a), (r1b, r2b, r3b))
        gsems = (gsa, gsb)
        wsems = (wsa, wsb)
        nch = _BPW * _S // _CH

        def start_g(c):
            s = c % 2
            bj, h = divmod(c * _CH, _S)
            idx = (i1b.at[bj, pl.ds(h, _CH)], i2b.at[bj, pl.ds(h, _CH)],
                   i3b.at[bj, pl.ds(h, _CH)])
            return tuple(
                pltpu.async_copy(f.at[ix], buf, gsems[s])
                for f, ix, buf in zip((f1, f2, f3), idx, bufs[s]))

        def start_w(c):
            s = c % 2
            row = b0 * _S + c * _CH
            return tuple(
                pltpu.async_copy(buf, g.at[pl.ds(row, _CH)], wsems[s])
                for g, buf in zip((g1, g2, g3), bufs[s]))

        pg = {0: start_g(0)}
        pw = {}
        for c in range(1, nch + 1):
            if c <= nch - 1:
                if c >= 2:
                    for w in pw.pop(c - 2):
                        w.wait()
                pg[c] = start_g(c)
            for g in pg.pop(c - 1):
                g.wait()
            pw[c - 1] = start_w(c - 1)
        for w in pw[nch - 2]:
            w.wait()
        for w in pw[nch - 1]:
            w.wait()

    return body


def _mm(x, w):
    return lax.dot_general(x, w, (((1,), (1,)), ((), ())),
                           preferred_element_type=jnp.float32)


def _a0_body(i04b, f0tb, w04, b04, g0o, st0):
    i = pl.program_id(0)

    @pl.when(i == 0)
    def _init():
        st0[...] = jnp.zeros_like(st0)

    idx = i04b[...]
    f0t = f0tb[...].astype(jnp.bfloat16)
    parts = []
    for j in range(_ABLK):
        row = idx[j:j + 1, :]
        n_iota = lax.broadcasted_iota(jnp.int32, (_N0, _S), 0)
        oh = (n_iota == row).astype(jnp.bfloat16)
        parts.append(lax.dot_general(
            oh, f0t[j], (((0,), (1,)), ((), ())),
            preferred_element_type=jnp.float32))
    g0 = jnp.concatenate(parts, axis=0)
    g0o[...] = g0
    z = _mm(g0, w04[...]) + b04[...]
    st0[0:1, :] += jnp.sum(z, axis=0, keepdims=True)
    st0[1:2, :] += jnp.sum(z * z, axis=0, keepdims=True)


def _norm_leaky(z, sm, sq):
    mu = sm * (1.0 / _M)
    ex2 = sq * (1.0 / _M)
    rs = lax.rsqrt(ex2 - mu * mu + _EPS)
    y = (z - mu) * rs
    return jnp.where(y > 0, y, 0.2 * y)


def _dense_body(st0, g0b, g1b, g2b, g3b, f4b,
                w04, w14, w24, w34, b04, b14, b24, b34, w4, b4,
                ob,
                sg1, sg2, sg3, sf4, sz5, st, st5, stage, sem):
    p = pl.program_id(0)
    i = pl.program_id(1)
    r = pl.ds(i * _BLK, _BLK)

    @pl.when((p == 0) & (i == 0))
    def _init():
        st[...] = jnp.zeros_like(st)
        st5[...] = jnp.zeros_like(st5)

    @pl.when(p == 0)
    def _phase0():
        for k, (gv, sg, w, bb) in enumerate(
                ((g1b[...], sg1, w14, b14),
                 (g2b[...], sg2, w24, b24), (g3b[...], sg3, w34, b34))):
            sg[r, :] = gv.astype(jnp.bfloat16)
            z = _mm(gv, w[...]) + bb[...]
            st[k:k + 1, :] += jnp.sum(z, axis=0, keepdims=True)
            st[k + 3:k + 4, :] += jnp.sum(z * z, axis=0, keepdims=True)

    @pl.when(p == 1)
    def _phase1():
        st0v = st0[...]
        stv = st[...]
        f4v = f4b[...]
        sf4[r, :] = f4v.astype(jnp.bfloat16)
        acc = f4v
        branches = ((g0b[...], w04, b04,
                     st0v[0:1, :], st0v[1:2, :]),
                    (sg1[r, :].astype(jnp.float32), w14, b14,
                     stv[0:1, :], stv[3:4, :]),
                    (sg2[r, :].astype(jnp.float32), w24, b24,
                     stv[1:2, :], stv[4:5, :]),
                    (sg3[r, :].astype(jnp.float32), w34, b34,
                     stv[2:3, :], stv[5:6, :]))
        for gv, w, bb, sm, sq in branches:
            z = _mm(gv, w[...]) + bb[...]
            acc = acc + _norm_leaky(z, sm, sq)
        z = _mm(acc, w4[...]) + b4[...]
        sz5[r, :] = z.astype(jnp.bfloat16)
        st5[0:1, :] += jnp.sum(z, axis=0, keepdims=True)
        st5[1:2, :] += jnp.sum(z * z, axis=0, keepdims=True)

    @pl.when(p == 2)
    def _phase2():
        stv = st5[...]
        stage[...] = (_norm_leaky(sz5[r, :].astype(jnp.float32),
                                  stv[0:1, :], stv[1:2, :])
                      + sf4[r, :].astype(jnp.float32))
        pltpu.async_copy(stage, ob.at[r, :], sem).wait()


def _rows(c, blk=_BLK):
    return pl.BlockSpec((blk, c), lambda i: (i, 0))


def _rep(shape):
    return pl.BlockSpec(shape, lambda i: tuple(0 for _ in shape))


_ARB = pltpu.CompilerParams(dimension_semantics=("arbitrary",))


def kernel(num_point, f0, f1, f2, f3, f4, FPS_0, FPS_1, FPS_2, FPS_3,
           W04, b04, W14, b14, W24, b24, W34, b34, W4, b4):
    i04, o1, o2, o3 = _sc_chain()(FPS_0, FPS_1, FPS_2, FPS_3)
    g1, g2, g3 = _sc_gather()(
        o1, o2, o3,
        f1.reshape(_B * _N1, _C1),
        f2.reshape(_B * _N2, _C2), f3.reshape(_B * _N3, _C3))

    b04r, b14r = b04.reshape(1, _C4), b14.reshape(1, _C4)
    b24r, b34r = b24.reshape(1, _C4), b34.reshape(1, _C4)
    b4r = b4.reshape(1, _C4)
    f4f = f4.reshape(_M, _C4)
    f0t = jnp.transpose(f0, (0, 2, 1))
    arows = _ABLK * _S

    g0, st0 = pl.pallas_call(
        _a0_body,
        grid=(_B // _ABLK,),
        in_specs=[_rows(_S, _ABLK),
                  pl.BlockSpec((_ABLK, _C0, _N0), lambda i: (i, 0, 0)),
                  _rep((_C4, _C0)), _rep((1, _C4))],
        out_specs=(_rows(_C0, arows), _rep((2, _C4))),
        out_shape=(jax.ShapeDtypeStruct((_M, _C0), jnp.float32),
                   jax.ShapeDtypeStruct((2, _C4), jnp.float32)),
        compiler_params=pltpu.CompilerParams(
            dimension_semantics=("arbitrary",),
            vmem_limit_bytes=100 * 1024 * 1024),
    )(i04, f0t, W04, b04r)

    def _prow(c, cond):
        return pl.BlockSpec(
            (_BLK, c), lambda p, i: (jnp.where(cond(p), i, 0), 0))

    def _prep(shape):
        return pl.BlockSpec(shape, lambda p, i: tuple(0 for _ in shape))

    out = pl.pallas_call(
        _dense_body,
        grid=(3, _M // _BLK),
        in_specs=[_prep((2, _C4)),
                  _prow(_C0, lambda p: p == 1),
                  _prow(_C1, lambda p: p == 0),
                  _prow(_C2, lambda p: p == 0),
                  _prow(_C3, lambda p: p == 0),
                  _prow(_C4, lambda p: p == 1),
                  _prep((_C4, _C0)), _prep((_C4, _C1)),
                  _prep((_C4, _C2)), _prep((_C4, _C3)),
                  _prep((1, _C4)), _prep((1, _C4)),
                  _prep((1, _C4)), _prep((1, _C4)),
                  _prep((_C4, _C4)), _prep((1, _C4))],
        out_specs=pl.BlockSpec(memory_space=pl.ANY),
        out_shape=jax.ShapeDtypeStruct((_M, _C4), jnp.float32),
        scratch_shapes=[
            pltpu.VMEM((_M, _C1), jnp.bfloat16),
            pltpu.VMEM((_M, _C2), jnp.bfloat16),
            pltpu.VMEM((_M, _C3), jnp.bfloat16),
            pltpu.VMEM((_M, _C4), jnp.bfloat16),
            pltpu.VMEM((_M, _C4), jnp.bfloat16),
            pltpu.VMEM((6, _C4), jnp.float32),
            pltpu.VMEM((2, _C4), jnp.float32),
            pltpu.VMEM((_BLK, _C4), jnp.float32),
            pltpu.SemaphoreType.DMA,
        ],
        compiler_params=pltpu.CompilerParams(
            dimension_semantics=("arbitrary", "arbitrary"),
            vmem_limit_bytes=120 * 1024 * 1024),
    )(st0, g0, g1, g2, g3, f4f, W04, W14, W24, W34,
      b04r, b14r, b24r, b34r, W4, b4r)

    return out.reshape(_B, _S, _C4)


# speedup vs baseline: 1.4112x; 1.0050x over previous
import functools

import jax
import jax.numpy as jnp
from jax import lax
from jax.experimental import pallas as pl
from jax.experimental.pallas import tpu as pltpu
from jax.experimental.pallas import tpu_sc as plsc

_B = 128
_S = 128
_N0, _N1, _N2, _N3 = 2048, 1024, 512, 256
_C0, _C1, _C2, _C3, _C4 = 64, 128, 128, 256, 256
_M = _B * _S
_EPS = 1e-5
_NC, _NS = 2, 16
_NW = _NC * _NS
_BPW = _B // _NW
_ABLK = 16
_BLK = 2048
_CH = 64

_SC_PARAMS = pltpu.CompilerParams(needs_layout_passes=False)


def _sc_mesh():
    return plsc.VectorSubcoreMesh(
        core_axis_name="c", subcore_axis_name="s",
        num_cores=_NC, num_subcores=_NS)


def _sc_chain():
    @functools.partial(
        pl.kernel,
        mesh=_sc_mesh(),
        compiler_params=_SC_PARAMS,
        out_type=(
            jax.ShapeDtypeStruct((_B, _S), jnp.int32),
            jax.ShapeDtypeStruct((_B, _S), jnp.int32),
            jax.ShapeDtypeStruct((_B, _S), jnp.int32),
            jax.ShapeDtypeStruct((_B, _S), jnp.int32),
        ),
        scratch_types=[
            pltpu.VMEM((_BPW, _N1), jnp.int32),
            pltpu.VMEM((_BPW, _N2), jnp.int32),
            pltpu.VMEM((_BPW, _N3), jnp.int32),
            pltpu.VMEM((_BPW, _S), jnp.int32),
            pltpu.VMEM((_BPW, _S), jnp.int32),
            pltpu.VMEM((_BPW, _S), jnp.int32),
            pltpu.VMEM((_BPW, _S), jnp.int32),
            pltpu.VMEM((_BPW, _S), jnp.int32),
        ],
    )
    def body(fps0, fps1, fps2, fps3,
             i04, o1, o2, o3,
             t0, t1, t2, t3, li0, gi1, gi2, gi3):
        wid = lax.axis_index("s") * _NC + lax.axis_index("c")
        b0 = wid * _BPW
        rows = pl.ds(b0, _BPW)
        pltpu.sync_copy(fps0.at[rows], t0)
        pltpu.sync_copy(fps1.at[rows], t1)
        pltpu.sync_copy(fps2.at[rows], t2)
        pltpu.sync_copy(fps3.at[rows], t3)

        for j in range(_BPW):
            jv = jnp.full((16,), j, jnp.int32)

            def chain(i, c, j=j, jv=jv):
                s = pl.ds(i * 16, 16)
                v3 = t3[j, s]
                v2 = plsc.load_gather(t2, [jv, v3])
                v1 = plsc.load_gather(t1, [jv, v2])
                v0 = plsc.load_gather(t0, [jv, v1])
                li0[j, s] = v0
                gi3[j, s] = v3 + (b0 + j) * _N3
                gi2[j, s] = v2 + (b0 + j) * _N2
                gi1[j, s] = v1 + (b0 + j) * _N1
                return c

            lax.fori_loop(0, _S // 16, chain, jnp.int32(0))

        pltpu.sync_copy(li0, i04.at[rows])
        pltpu.sync_copy(gi1, o1.at[rows])
        pltpu.sync_copy(gi2, o2.at[rows])
        pltpu.sync_copy(gi3, o3.at[rows])

    return body


def _sc_gather():
    @functools.partial(
        pl.kernel,
        mesh=_sc_mesh(),
        compiler_params=_SC_PARAMS,
        out_type=(
            jax.ShapeDtypeStruct((_M, _C1), jnp.float32),
            jax.ShapeDtypeStruct((_M, _C2), jnp.float32),
            jax.ShapeDtypeStruct((_M, _C3), jnp.float32),
        ),
        scratch_types=[
            pltpu.VMEM((_BPW, _S), jnp.int32),
            pltpu.VMEM((_BPW, _S), jnp.int32),
            pltpu.VMEM((_BPW, _S), jnp.int32),
            pltpu.VMEM((_CH, _C1), jnp.float32),
            pltpu.VMEM((_CH, _C2), jnp.float32),
            pltpu.VMEM((_CH, _C3), jnp.float32),
            pltpu.VMEM((_CH, _C1), jnp.float32),
            pltpu.VMEM((_CH, _C2), jnp.float32),
            pltpu.VMEM((_CH, _C3), jnp.float32),
            pltpu.SemaphoreType.DMA,
            pltpu.SemaphoreType.DMA,
            pltpu.SemaphoreType.DMA,
            pltpu.SemaphoreType.DMA,
        ],
    )
    def body(o1, o2, o3, f1, f2, f3,
             g1, g2, g3,
             i1b, i2b, i3b, r1a, r2a, r3a, r1b, r2b, r3b,
             gsa, gsb, wsa, wsb):
        wid = lax.axis_index("s") * _NC + lax.axis_index("c")
        b0 = wid * _BPW
        rows = pl.ds(b0, _BPW)
        pltpu.sync_copy(o1.at[rows], i1b)
        pltpu.sync_copy(o2.at[rows], i2b)
        pltpu.sync_copy(o3.at[rows], i3b)

        bufs = ((r1a, r2a, r3a), (r1b, r2b, r3b))
        gsems = (gsa, gsb)
        wsems = (wsa, wsb)
        nch = _BPW * _S // _CH

        def start_g(c):
            s = c % 2
            bj, h = divmod(c * _CH, _S)
            idx = (i1b.at[bj, pl.ds(h, _CH)], i2b.at[bj, pl.ds(h, _CH)],
                   i3b.at[bj, pl.ds(h, _CH)])
            return tuple(
                pltpu.async_copy(f.at[ix], buf, gsems[s])
                for f, ix, buf in zip((f1, f2, f3), idx, bufs[s]))

        def start_w(c):
            s = c % 2
            row = b0 * _S + c * _CH
            return tuple(
                pltpu.async_copy(buf, g.at[pl.ds(row, _CH)], wsems[s])
                for g, buf in zip((g1, g2, g3), bufs[s]))

        pg = {0: start_g(0)}
        pw = {}
        for c in range(1, nch + 1):
            if c <= nch - 1:
                if c >= 2:
                    for w in pw.pop(c - 2):
                        w.wait()
                pg[c] = start_g(c)
            for g in pg.pop(c - 1):
                g.wait()
            pw[c - 1] = start_w(c - 1)
        for w in pw[nch - 2]:
            w.wait()
        for w in pw[nch - 1]:
            w.wait()

    return body


def _mm(x, w):
    return lax.dot_general(x, w, (((1,), (1,)), ((), ())),
                           preferred_element_type=jnp.float32)


def _a0_body(i04b, f0tb, w04, b04, g0o, st0):
    i = pl.program_id(0)

    @pl.when(i == 0)
    def _init():
        st0[...] = jnp.zeros_like(st0)

    idx = i04b[...]
    f0t = f0tb[...].astype(jnp.bfloat16)
    parts = []
    for j in range(_ABLK):
        row = idx[j:j + 1, :]
        n_iota = lax.broadcasted_iota(jnp.int32, (_N0, _S), 0)
        oh = (n_iota == row).astype(jnp.bfloat16)
        parts.append(lax.dot_general(
            oh, f0t[j], (((0,), (1,)), ((), ())),
            preferred_element_type=jnp.float32))
    g0 = jnp.concatenate(parts, axis=0)
    g0o[...] = g0
    z = _mm(g0, w04[...]) + b04[...]
    st0[0:1, :] += jnp.sum(z, axis=0, keepdims=True)
    st0[1:2, :] += jnp.sum(z * z, axis=0, keepdims=True)


def _norm_leaky(z, sm, sq):
    mu = sm * (1.0 / _M)
    ex2 = sq * (1.0 / _M)
    rs = lax.rsqrt(ex2 - mu * mu + _EPS)
    y = (z - mu) * rs
    return jnp.where(y > 0, y, 0.2 * y)


def _dense_body(st0, g0b, g1b, g2b, g3b, f4b,
                w04, w14, w24, w34, b04, b14, b24, b34, w4, b4,
                ob,
                sg1, sg2, sg3, sf4, sz5, st, st5, stage, sem):
    p = pl.program_id(0)
    i = pl.program_id(1)
    r = pl.ds(i * _BLK, _BLK)

    @pl.when((p == 0) & (i == 0))
    def _init():
        st[...] = jnp.zeros_like(st)
        st5[...] = jnp.zeros_like(st5)

    @pl.when(p == 0)
    def _phase0():
        for k, (gv, sg, w, bb) in enumerate(
                ((g1b[...], sg1, w14, b14),
                 (g2b[...], sg2, w24, b24), (g3b[...], sg3, w34, b34))):
            sg[r, :] = gv.astype(jnp.bfloat16)
            z = _mm(gv, w[...]) + bb[...]
            st[k:k + 1, :] += jnp.sum(z, axis=0, keepdims=True)
            st[k + 3:k + 4, :] += jnp.sum(z * z, axis=0, keepdims=True)

    @pl.when(p == 1)
    def _phase1():
        st0v = st0[...]
        stv = st[...]
        f4v = f4b[...]
        sf4[r, :] = f4v.astype(jnp.bfloat16)
        acc = f4v
        branches = ((g0b[...].astype(jnp.bfloat16), w04, b04,
                     st0v[0:1, :], st0v[1:2, :]),
                    (sg1[r, :], w14, b14,
                     stv[0:1, :], stv[3:4, :]),
                    (sg2[r, :], w24, b24,
                     stv[1:2, :], stv[4:5, :]),
                    (sg3[r, :], w34, b34,
                     stv[2:3, :], stv[5:6, :]))
        for gv, w, bb, sm, sq in branches:
            z = _mm(gv, w[...].astype(jnp.bfloat16)) + bb[...]
            acc = acc + _norm_leaky(z, sm, sq)
        z = _mm(acc.astype(jnp.bfloat16), w4[...].astype(jnp.bfloat16))
        z = z + b4[...]
        sz5[r, :] = z.astype(jnp.bfloat16)
        st5[0:1, :] += jnp.sum(z, axis=0, keepdims=True)
        st5[1:2, :] += jnp.sum(z * z, axis=0, keepdims=True)

    @pl.when(p == 2)
    def _phase2():
        stv = st5[...]
        stage[...] = (_norm_leaky(sz5[r, :].astype(jnp.float32),
                                  stv[0:1, :], stv[1:2, :])
                      + sf4[r, :].astype(jnp.float32))
        pltpu.async_copy(stage, ob.at[r, :], sem).wait()


def _rows(c, blk=_BLK):
    return pl.BlockSpec((blk, c), lambda i: (i, 0))


def _rep(shape):
    return pl.BlockSpec(shape, lambda i: tuple(0 for _ in shape))


_ARB = pltpu.CompilerParams(dimension_semantics=("arbitrary",))


def kernel(num_point, f0, f1, f2, f3, f4, FPS_0, FPS_1, FPS_2, FPS_3,
           W04, b04, W14, b14, W24, b24, W34, b34, W4, b4):
    i04, o1, o2, o3 = _sc_chain()(FPS_0, FPS_1, FPS_2, FPS_3)
    g1, g2, g3 = _sc_gather()(
        o1, o2, o3,
        f1.reshape(_B * _N1, _C1),
        f2.reshape(_B * _N2, _C2), f3.reshape(_B * _N3, _C3))

    b04r, b14r = b04.reshape(1, _C4), b14.reshape(1, _C4)
    b24r, b34r = b24.reshape(1, _C4), b34.reshape(1, _C4)
    b4r = b4.reshape(1, _C4)
    f4f = f4.reshape(_M, _C4)
    f0t = jnp.transpose(f0, (0, 2, 1))
    arows = _ABLK * _S

    g0, st0 = pl.pallas_call(
        _a0_body,
        grid=(_B // _ABLK,),
        in_specs=[_rows(_S, _ABLK),
                  pl.BlockSpec((_ABLK, _C0, _N0), lambda i: (i, 0, 0)),
                  _rep((_C4, _C0)), _rep((1, _C4))],
        out_specs=(_rows(_C0, arows), _rep((2, _C4))),
        out_shape=(jax.ShapeDtypeStruct((_M, _C0), jnp.float32),
                   jax.ShapeDtypeStruct((2, _C4), jnp.float32)),
        compiler_params=pltpu.CompilerParams(
            dimension_semantics=("arbitrary",),
            vmem_limit_bytes=100 * 1024 * 1024),
    )(i04, f0t, W04, b04r)

    def _prow(c, cond):
        return pl.BlockSpec(
            (_BLK, c), lambda p, i: (jnp.where(cond(p), i, 0), 0))

    def _prep(shape):
        return pl.BlockSpec(shape, lambda p, i: tuple(0 for _ in shape))

    out = pl.pallas_call(
        _dense_body,
        grid=(3, _M // _BLK),
        in_specs=[_prep((2, _C4)),
                  _prow(_C0, lambda p: p == 1),
                  _prow(_C1, lambda p: p == 0),
                  _prow(_C2, lambda p: p == 0),
                  _prow(_C3, lambda p: p == 0),
                  _prow(_C4, lambda p: p == 1),
                  _prep((_C4, _C0)), _prep((_C4, _C1)),
                  _prep((_C4, _C2)), _prep((_C4, _C3)),
                  _prep((1, _C4)), _prep((1, _C4)),
                  _prep((1, _C4)), _prep((1, _C4)),
                  _prep((_C4, _C4)), _prep((1, _C4))],
        out_specs=pl.BlockSpec(memory_space=pl.ANY),
        out_shape=jax.ShapeDtypeStruct((_M, _C4), jnp.float32),
        scratch_shapes=[
            pltpu.VMEM((_M, _C1), jnp.bfloat16),
            pltpu.VMEM((_M, _C2), jnp.bfloat16),
            pltpu.VMEM((_M, _C3), jnp.bfloat16),
            pltpu.VMEM((_M, _C4), jnp.bfloat16),
            pltpu.VMEM((_M, _C4), jnp.bfloat16),
            pltpu.VMEM((6, _C4), jnp.float32),
            pltpu.VMEM((2, _C4), jnp.float32),
            pltpu.VMEM((_BLK, _C4), jnp.float32),
            pltpu.SemaphoreType.DMA,
        ],
        compiler_params=pltpu.CompilerParams(
            dimension_semantics=("arbitrary", "arbitrary"),
            vmem_limit_bytes=120 * 1024 * 1024),
    )(st0, g0, g1, g2, g3, f4f, W04, W14, W24, W34,
      b04r, b14r, b24r, b34r, W4, b4r)

    return out.reshape(_B, _S, _C4)
